# baseline (speedup 1.0000x reference)
import numpy as np
import jax
import jax.numpy as jnp
from jax.experimental import pallas as pl

HEADS = 8
HD = 48
C = HEADS * HD
N = 1024
NH = 512
MH = 32
MS = HEADS * MH

_BF = jnp.bfloat16
_F32 = jnp.float32


def _bdot(a, b):
    return jnp.dot(a.astype(_BF), b.astype(_BF), preferred_element_type=_F32)


def _constants():
    nh = np.arange(NH)
    w, h = nh // 32, nh % 32
    s_of_n = (h // 16) * 16 + (w // 4) * 4 + (h % 16) // 4
    ss = np.arange(MH)
    pool = ((ss[None, :] == s_of_n[:, None]) / 16.0).astype(np.float32)
    validh = (ss[:, None] // 16 == (h // 16)[None, :])
    validh = np.tile(validh, (HEADS, 1)).astype(np.float32)
    pooth = np.tile(pool.T, (HEADS, 1)).astype(np.float32)
    cc = np.arange(C)
    jj = np.arange(MS)
    bdh = (cc[:, None] // HD == jj[None, :] // MH).astype(np.float32)
    return jnp.asarray(pool), jnp.asarray(validh), jnp.asarray(pooth), \
        jnp.asarray(bdh)


def _cluster_kernel(x_ref, wf_ref, bf_ref, wv_ref, bv_ref, wp_ref, bp_ref,
                    ab_ref, pool_ref, validh_ref, pooth_ref, bdh_ref,
                    out_ref):
    xmat = x_ref[0]
    xf = _bdot(wf_ref[...], xmat) + bf_ref[...]
    val = _bdot(wv_ref[...], xmat) + bv_ref[...]

    ab = ab_ref[...]
    alpha = ab[:, 0:1]
    beta = ab[:, 1:2]
    pooth = pooth_ref[...]
    bdh = bdh_ref[...]

    xpool = _bdot(xmat, pool_ref[...])
    wfb = wf_ref[...].astype(_BF)
    h1 = xpool.astype(_BF)
    r1 = xpool - h1.astype(_F32)
    h2 = r1.astype(_BF)
    h3 = (r1 - h2.astype(_F32)).astype(_BF)
    cen = (jnp.dot(wfb, h1, preferred_element_type=_F32)
           + jnp.dot(wfb, h2, preferred_element_type=_F32)
           + jnp.dot(wfb, h3, preferred_element_type=_F32)
           + bf_ref[...])

    xf3 = xf.reshape(HEADS, HD, NH)
    xfn = (xf3 * (1.0 / jnp.maximum(
        jnp.sqrt(jnp.sum(xf3 * xf3, axis=1, keepdims=True)), 1e-12))
           ).reshape(C, NH)
    cen3 = cen.reshape(HEADS, HD, MH)
    cenn = (cen3 * (1.0 / jnp.maximum(
        jnp.sqrt(jnp.sum(cen3 * cen3, axis=1, keepdims=True)), 1e-12))
            ).reshape(C, MH)

    cen_bd = jnp.tile(cenn, (1, HEADS)) * bdh
    sim = jax.nn.sigmoid(
        beta + alpha * jnp.einsum('cm,cn->mn', cen_bd.astype(_BF),
                                  xfn.astype(_BF),
                                  preferred_element_type=_F32))

    mi = jax.lax.broadcasted_iota(jnp.int32, (1, MH, NH), 1)
    simv = jnp.where(validh_ref[...] > 0.5, sim, -1.0).reshape(HEADS, MH, NH)
    amax = jnp.max(simv, axis=1, keepdims=True)
    first = jnp.min(jnp.where(simv >= amax, mi, MH), axis=1, keepdims=True)
    simm = (jnp.where(mi == first, sim.reshape(HEADS, MH, NH), 0.0)
            ).reshape(MS, NH)

    simmp = (simm + pooth).astype(_BF)
    aggvc = jnp.einsum('cn,mn->cm', val.astype(_BF), simmp,
                       preferred_element_type=_F32)
    denom1 = jnp.einsum('xn,mn->xm', jnp.full((1, NH), 1.0, _BF), simmp,
                        preferred_element_type=_F32)

    out_m = aggvc * ((1.0 / denom1) * bdh)
    merged = _bdot(out_m, simm)
    out_ref[0] = _bdot(wp_ref[...], merged) + bp_ref[...]


def kernel(x, Wf, bf, Wv, bv, Wp, bp, sim_alpha, sim_beta):
    B = x.shape[0]
    x2 = x.reshape(B, C, N)
    ab = jnp.concatenate([sim_alpha, sim_beta]).reshape(1, 2)
    bf2 = bf.reshape(C, 1)
    bv2 = bv.reshape(C, 1)
    bp2 = bp.reshape(C, 1)
    pool, validh, pooth, bdh = _constants()

    fixed = lambda b, f: (0, 0)
    out = pl.pallas_call(
        _cluster_kernel,
        grid=(B, 2),
        in_specs=[
            pl.BlockSpec((1, C, NH), lambda b, f: (b, 0, f)),
            pl.BlockSpec((C, C), fixed),
            pl.BlockSpec((C, 1), fixed),
            pl.BlockSpec((C, C), fixed),
            pl.BlockSpec((C, 1), fixed),
            pl.BlockSpec((C, C), fixed),
            pl.BlockSpec((C, 1), fixed),
            pl.BlockSpec((1, 2), fixed),
            pl.BlockSpec((NH, MH), fixed),
            pl.BlockSpec((MS, NH), fixed),
            pl.BlockSpec((MS, NH), fixed),
            pl.BlockSpec((C, MS), fixed),
        ],
        out_specs=pl.BlockSpec((1, C, NH), lambda b, f: (b, 0, f)),
        out_shape=jax.ShapeDtypeStruct((B, C, N), jnp.float32),
    )(x2, Wf, bf2, Wv, bv2, Wp, bp2, ab, pool, validh, pooth, bdh)

    return out.reshape(B, C, 32, 32)


# speedup vs baseline: 1.1152x; 1.1152x over previous
import numpy as np
import jax
import jax.numpy as jnp
from jax.experimental import pallas as pl

HEADS = 8
HD = 48
C = HEADS * HD
N = 1024
NH = 512
M = 64
MH = 32
MS = HEADS * MH

_BF = jnp.bfloat16
_F32 = jnp.float32


def _bdot(a, b):
    return jnp.dot(a.astype(_BF), b.astype(_BF), preferred_element_type=_F32)


def _constants():
    n = np.arange(N)
    w, h = n // 32, n % 32
    quad = (w // 16) * 2 + (h // 16)
    m_of_n = quad * 16 + ((w % 16) // 4) * 4 + ((h % 16) // 4)
    mm = np.arange(M)
    pool = ((mm[None, :] == m_of_n[:, None]) / 16.0).astype(np.float32)

    s = np.arange(MH)
    nh = np.arange(NH)
    validh = (s[:, None] // 16 == (nh[None, :] % 32) // 16)
    validh = np.tile(validh, (HEADS, 1)).astype(np.float32)
    pooth = np.tile(pool[:NH, :MH].T, (HEADS, 1)).astype(np.float32)
    cc = np.arange(C)
    jj = np.arange(MS)
    bdh = (cc[:, None] // HD == jj[None, :] // MH).astype(np.float32)
    return jnp.asarray(pool), jnp.asarray(validh), jnp.asarray(pooth), \
        jnp.asarray(bdh)


def _cluster_kernel(x_ref, wf_ref, bf_ref, wv_ref, bv_ref, wp_ref, bp_ref,
                    ab_ref, pool_ref, validh_ref, pooth_ref, bdh_ref,
                    out_ref):
    xmat = x_ref[0]
    xf = _bdot(wf_ref[...], xmat) + bf_ref[...]
    val = _bdot(wv_ref[...], xmat) + bv_ref[...]

    ab = ab_ref[...]
    alpha = ab[:, 0:1]
    beta = ab[:, 1:2]
    pooth = pooth_ref[...]
    bdh = bdh_ref[...]

    xpool = _bdot(xmat, pool_ref[...])
    wfb = wf_ref[...].astype(_BF)
    h1 = xpool.astype(_BF)
    r1 = xpool - h1.astype(_F32)
    h2 = r1.astype(_BF)
    h3 = (r1 - h2.astype(_F32)).astype(_BF)
    cen_all = (jnp.dot(wfb, h1, preferred_element_type=_F32)
               + jnp.dot(wfb, h2, preferred_element_type=_F32)
               + jnp.dot(wfb, h3, preferred_element_type=_F32)
               + bf_ref[...])

    xf3 = xf.reshape(HEADS, HD, N)
    xfn = (xf3 * (1.0 / jnp.maximum(
        jnp.sqrt(jnp.sum(xf3 * xf3, axis=1, keepdims=True)), 1e-12))
           ).reshape(C, N)
    cen3 = cen_all.reshape(HEADS, HD, M)
    cenn = (cen3 * (1.0 / jnp.maximum(
        jnp.sqrt(jnp.sum(cen3 * cen3, axis=1, keepdims=True)), 1e-12))
            ).reshape(C, M)

    vmask = validh_ref[...] > 0.5
    mi = jax.lax.broadcasted_iota(jnp.int32, (1, MH, NH), 1)
    for f1 in range(2):
        xfn_h = jax.lax.slice(xfn, (0, NH * f1), (C, NH * (f1 + 1)))
        val_h = jax.lax.slice(val, (0, NH * f1), (C, NH * (f1 + 1)))
        cen_s = jax.lax.slice(cenn, (0, MH * f1), (C, MH * (f1 + 1)))

        cen_bd = jnp.tile(cen_s, (1, HEADS)) * bdh
        sim = jax.nn.sigmoid(
            beta + alpha * jnp.einsum('cm,cn->mn', cen_bd.astype(_BF),
                                      xfn_h.astype(_BF),
                                      preferred_element_type=_F32))

        simv = jnp.where(vmask, sim, -1.0).reshape(HEADS, MH, NH)
        amax = jnp.max(simv, axis=1, keepdims=True)
        first = jnp.min(jnp.where(simv >= amax, mi, MH), axis=1,
                        keepdims=True)
        simm = (jnp.where(mi == first, sim.reshape(HEADS, MH, NH), 0.0)
                ).reshape(MS, NH)

        simmp = (simm + pooth).astype(_BF)
        aggvc = jnp.einsum('cn,mn->cm', val_h.astype(_BF), simmp,
                           preferred_element_type=_F32)
        denom1 = jnp.einsum('xn,mn->xm',
                            jnp.full((1, NH), 1.0, _BF), simmp,
                            preferred_element_type=_F32)

        out_m = aggvc * ((1.0 / denom1) * bdh)
        merged = _bdot(out_m, simm)
        fin = _bdot(wp_ref[...], merged) + bp_ref[...]
        out_ref[0, :, NH * f1:NH * (f1 + 1)] = fin


def kernel(x, Wf, bf, Wv, bv, Wp, bp, sim_alpha, sim_beta):
    B = x.shape[0]
    x2 = x.reshape(B, C, N)
    ab = jnp.concatenate([sim_alpha, sim_beta]).reshape(1, 2)
    bf2 = bf.reshape(C, 1)
    bv2 = bv.reshape(C, 1)
    bp2 = bp.reshape(C, 1)
    pool, validh, pooth, bdh = _constants()

    fixed = lambda b: (0, 0)
    out = pl.pallas_call(
        _cluster_kernel,
        grid=(B,),
        in_specs=[
            pl.BlockSpec((1, C, N), lambda b: (b, 0, 0)),
            pl.BlockSpec((C, C), fixed),
            pl.BlockSpec((C, 1), fixed),
            pl.BlockSpec((C, C), fixed),
            pl.BlockSpec((C, 1), fixed),
            pl.BlockSpec((C, C), fixed),
            pl.BlockSpec((C, 1), fixed),
            pl.BlockSpec((1, 2), fixed),
            pl.BlockSpec((N, M), fixed),
            pl.BlockSpec((MS, NH), fixed),
            pl.BlockSpec((MS, NH), fixed),
            pl.BlockSpec((C, MS), fixed),
        ],
        out_specs=pl.BlockSpec((1, C, N), lambda b: (b, 0, 0)),
        out_shape=jax.ShapeDtypeStruct((B, C, N), jnp.float32),
    )(x2, Wf, bf2, Wv, bv2, Wp, bp2, ab, pool, validh, pooth, bdh)

    return out.reshape(B, C, 32, 32)
